# Initial kernel scaffold; baseline (speedup 1.0000x reference)
import jax
import jax.numpy as jnp
from jax.experimental import pallas as pl


def kernel(adj, x, W1, W2, W3, b3, W4, b4):
    raise NotImplementedError("write your pallas kernel here")


# speedup vs baseline: 1.5285x; 1.5285x over previous
import numpy as np
import jax
import jax.numpy as jnp
from jax.experimental import pallas as pl
from jax.experimental.pallas import tpu as pltpu

N = 4096
F = 256
H = 128
ALPHA = 0.8
BM = 256
BT = 512
NBT = N // BT
_UPPER = np.array([(r, c) for r in range(NBT) for c in range(r, NBT)],
                  dtype=np.int32).T.copy()
NT = _UPPER.shape[1]

_ARB = pltpu.CompilerParams(dimension_semantics=("arbitrary",))
_F32 = jnp.float32


def _k1_body(x_ref, w1_ref, w3_ref, b3_ref, hx_ref, u3_ref):
    x = x_ref[...]
    hx_ref[...] = jnp.dot(x, w1_ref[...], preferred_element_type=_F32)
    u3_ref[...] = jnp.dot(x, w3_ref[...], preferred_element_type=_F32) + b3_ref[...]


def _k1(x, W1, W3, b3r):
    return pl.pallas_call(
        _k1_body,
        out_shape=[jax.ShapeDtypeStruct((N, H), _F32),
                   jax.ShapeDtypeStruct((N, H), _F32)],
    )(x, W1, W3, b3r)


def _k2_body(adj_ref, hx_ref, h1_ref, s_ref, ct_ref):
    i = pl.program_id(0)
    a = adj_ref[...]
    h1_ref[...] = jnp.dot(a, hx_ref[...], preferred_element_type=_F32)
    rowg = i * BM + jax.lax.broadcasted_iota(jnp.int32, (BM, N), 0)
    colg = jax.lax.broadcasted_iota(jnp.int32, (BM, N), 1)
    am = jnp.where(colg > rowg, a, 0.0)
    s_ref[...] = jnp.sum(am, axis=1, keepdims=True)

    @pl.when(i == 0)
    def _():
        ct_ref[...] = jnp.zeros_like(ct_ref)

    ct_ref[...] += jnp.sum(am, axis=0, keepdims=True)


def _k2(adj, hx):
    return pl.pallas_call(
        _k2_body,
        grid=(N // BM,),
        in_specs=[pl.BlockSpec((BM, N), lambda i: (i, 0)),
                  pl.BlockSpec((N, H), lambda i: (0, 0))],
        out_specs=[pl.BlockSpec((BM, H), lambda i: (i, 0)),
                   pl.BlockSpec((BM, 1), lambda i: (i, 0)),
                   pl.BlockSpec((1, N), lambda i: (0, 0))],
        out_shape=[jax.ShapeDtypeStruct((N, H), _F32),
                   jax.ShapeDtypeStruct((N, 1), _F32),
                   jax.ShapeDtypeStruct((1, N), _F32)],
        compiler_params=_ARB,
    )(adj, hx)


def _k3_body(adj_ref, h1_ref, w2_ref, z_ref, lii_ref, st_ref, g1):
    i = pl.program_id(0)

    @pl.when(i == 0)
    def _():
        g1[...] = jnp.dot(h1_ref[...], w2_ref[...], preferred_element_type=_F32)

    z = jnp.maximum(jnp.dot(adj_ref[...], g1[...], preferred_element_type=_F32), 0.0)
    z_ref[...] = z
    lii_ref[...] = jnp.sum(z * z, axis=1, keepdims=True)

    @pl.when(i == 0)
    def _():
        st_ref[...] = jnp.zeros_like(st_ref)

    st_ref[...] += jnp.sum(z, axis=0, keepdims=True)


def _k3(adj, h1, W2):
    return pl.pallas_call(
        _k3_body,
        grid=(N // BM,),
        in_specs=[pl.BlockSpec((BM, N), lambda i: (i, 0)),
                  pl.BlockSpec((N, H), lambda i: (0, 0)),
                  pl.BlockSpec((H, H), lambda i: (0, 0))],
        out_specs=[pl.BlockSpec((BM, H), lambda i: (i, 0)),
                   pl.BlockSpec((BM, 1), lambda i: (i, 0)),
                   pl.BlockSpec((1, H), lambda i: (0, 0))],
        out_shape=[jax.ShapeDtypeStruct((N, H), _F32),
                   jax.ShapeDtypeStruct((N, 1), _F32),
                   jax.ShapeDtypeStruct((1, H), _F32)],
        scratch_shapes=[pltpu.VMEM((N, H), _F32)],
        compiler_params=_ARB,
    )(adj, h1, W2)


def _k4_body(rc_ref, zr_ref, zct_ref, m_ref):
    t = pl.program_id(0)
    l = jnp.dot(zr_ref[...], zct_ref[...], preferred_element_type=_F32)
    tm = jnp.full((1, H), jnp.max(l), _F32)

    @pl.when(t == 0)
    def _():
        m_ref[...] = tm

    m_ref[...] = jnp.maximum(m_ref[...], tm)


def _k4(rc, z, zt):
    grid_spec = pltpu.PrefetchScalarGridSpec(
        num_scalar_prefetch=1,
        grid=(NT,),
        in_specs=[pl.BlockSpec((BT, H), lambda t, rc: (rc[0, t], 0)),
                  pl.BlockSpec((H, BT), lambda t, rc: (0, rc[1, t]))],
        out_specs=pl.BlockSpec((1, H), lambda t, rc: (0, 0)),
    )
    return pl.pallas_call(
        _k4_body,
        grid_spec=grid_spec,
        out_shape=jax.ShapeDtypeStruct((1, H), _F32),
        compiler_params=_ARB,
    )(rc, z, zt)


def _k5_body(m_ref, lii_ref, s_col_ref, s_ref, c_ref, z_ref, zt_ref, u3_ref,
             d_ref, du3_ref, t3s_ref, liia_ref):
    am = ALPHA / m_ref[...]
    zdots = jnp.sum(z_ref[...] * s_col_ref[...], axis=1, keepdims=True)
    rowsum = 1.0 + am * (zdots - lii_ref[...]) \
        + (1.0 - ALPHA) * (s_ref[...] + c_ref[...])
    d = jax.lax.rsqrt(rowsum)
    d_ref[...] = d
    du3 = d * u3_ref[...]
    du3_ref[...] = du3
    t3s_ref[...] = am * jnp.dot(zt_ref[...], du3, preferred_element_type=_F32)
    liia_ref[...] = am * lii_ref[...]


def _k5(m, lii, s_col, s, ccol, z, zt, u3):
    return pl.pallas_call(
        _k5_body,
        out_shape=[jax.ShapeDtypeStruct((N, H), _F32),
                   jax.ShapeDtypeStruct((N, H), _F32),
                   jax.ShapeDtypeStruct((H, H), _F32),
                   jax.ShapeDtypeStruct((N, H), _F32)],
    )(m, lii, s_col, s, ccol, z, zt, u3)


def _k6_body(rc_ref, adj_ref, dvc_ref, dvtr_ref, o1_ref, o2t_ref):
    t = pl.program_id(0)
    r = rc_ref[0, t]
    c = rc_ref[1, t]
    a = adj_ref[...]
    rowg = r * BT + jax.lax.broadcasted_iota(jnp.int32, (BT, BT), 0)
    colg = c * BT + jax.lax.broadcasted_iota(jnp.int32, (BT, BT), 1)
    am = jnp.where(colg > rowg, a, 0.0)

    @pl.when(t == 0)
    def _():
        o1_ref[...] = jnp.zeros_like(o1_ref)
        o2t_ref[...] = jnp.zeros_like(o2t_ref)

    o1_ref[r] += jnp.dot(am, dvc_ref[...], preferred_element_type=_F32)
    o2t_ref[c] += jnp.dot(dvtr_ref[...], am, preferred_element_type=_F32)


def _k6(rc, adj, dv, dvt):
    grid_spec = pltpu.PrefetchScalarGridSpec(
        num_scalar_prefetch=1,
        grid=(NT,),
        in_specs=[pl.BlockSpec((BT, BT), lambda t, rc: (rc[0, t], rc[1, t])),
                  pl.BlockSpec((BT, H), lambda t, rc: (rc[1, t], 0)),
                  pl.BlockSpec((H, BT), lambda t, rc: (0, rc[0, t]))],
        out_specs=[pl.BlockSpec((NBT, BT, H), lambda t, rc: (0, 0, 0)),
                   pl.BlockSpec((NBT, H, BT), lambda t, rc: (0, 0, 0))],
    )
    return pl.pallas_call(
        _k6_body,
        grid_spec=grid_spec,
        out_shape=[jax.ShapeDtypeStruct((NBT, BT, H), _F32),
                   jax.ShapeDtypeStruct((NBT, H, BT), _F32)],
        compiler_params=_ARB,
    )(rc, adj, dv, dvt)


def _k6b_body(o1_ref, o2_ref, du3_ref, z_ref, t3s_ref, liia_ref, d_ref,
              w4_ref, b4_ref, zt_ref, m_ref, dv2_ref, t4s_ref):
    du3 = du3_ref[...]
    core = jnp.dot(z_ref[...], t3s_ref[...], preferred_element_type=_F32) \
        - liia_ref[...] * du3 \
        + (1.0 - ALPHA) * (o1_ref[...] + o2_ref[...]) + du3
    h1c = jnp.maximum(d_ref[...] * core, 0.0)
    v2 = jnp.dot(h1c, w4_ref[...], preferred_element_type=_F32) + b4_ref[...]
    dv2 = d_ref[...] * v2
    dv2_ref[...] = dv2
    am = ALPHA / m_ref[...]
    t4s_ref[...] = am * jnp.dot(zt_ref[...], dv2, preferred_element_type=_F32)


def _k6b(o1, o2, du3, z, t3s, liia, d, W4p, b4p, zt, m):
    return pl.pallas_call(
        _k6b_body,
        out_shape=[jax.ShapeDtypeStruct((N, H), _F32),
                   jax.ShapeDtypeStruct((H, H), _F32)],
    )(o1, o2, du3, z, t3s, liia, d, W4p, b4p, zt, m)


def _k7b_body(o1_ref, o2_ref, dv2_ref, z_ref, t4s_ref, liia_ref, d_ref,
              out_ref):
    dv2 = dv2_ref[...]
    core = jnp.dot(z_ref[...], t4s_ref[...], preferred_element_type=_F32) \
        - liia_ref[...] * dv2 \
        + (1.0 - ALPHA) * (o1_ref[...] + o2_ref[...]) + dv2
    out_ref[...] = d_ref[...] * core


def _k7b(o1, o2, dv2, z, t4s, liia, d):
    return pl.pallas_call(
        _k7b_body,
        out_shape=jax.ShapeDtypeStruct((N, H), _F32),
    )(o1, o2, dv2, z, t4s, liia, d)


def kernel(adj, x, W1, W2, W3, b3, W4, b4):
    nclass = W4.shape[1]
    b3r = b3.reshape(1, H).astype(_F32)
    W4p = jnp.zeros((H, H), _F32).at[:, :nclass].set(W4)
    b4p = jnp.zeros((1, H), _F32).at[0, :nclass].set(b4)
    rc = jnp.asarray(_UPPER)

    hx, u3 = _k1(x, W1, W3, b3r)
    h1, s, cT = _k2(adj, hx)
    z, lii, s_col = _k3(adj, h1, W2)
    zt = z.T
    m = _k4(rc, z, zt)
    ccol = cT.T
    d, du3, t3s, liia = _k5(m, lii, s_col, s, ccol, z, zt, u3)
    du3t = du3.T
    o1a, o2ta = _k6(rc, adj, du3, du3t)
    o1 = o1a.reshape(N, H)
    o2 = jnp.transpose(o2ta, (0, 2, 1)).reshape(N, H)
    dv2, t4s = _k6b(o1, o2, du3, z, t3s, liia, d, W4p, b4p, zt, m)
    dv2t = dv2.T
    o1b, o2tb = _k6(rc, adj, dv2, dv2t)
    o1f = o1b.reshape(N, H)
    o2f = jnp.transpose(o2tb, (0, 2, 1)).reshape(N, H)
    outf = _k7b(o1f, o2f, dv2, z, t4s, liia, d)
    return outf[:, :nclass]


# speedup vs baseline: 1.7898x; 1.1709x over previous
import numpy as np
import jax
import jax.numpy as jnp
from jax.experimental import pallas as pl
from jax.experimental.pallas import tpu as pltpu

N = 4096
F = 256
H = 128
ALPHA = 0.8
BM = 256
BT = 512
NBT = N // BT
_UPPER = np.array([(r, c) for r in range(NBT) for c in range(r, NBT)],
                  dtype=np.int32).T.copy()
NT = _UPPER.shape[1]

_ARB = pltpu.CompilerParams(dimension_semantics=("arbitrary",))
_F32 = jnp.float32
_BF16 = jnp.bfloat16


def _k1_body(x_ref, w1_ref, w3_ref, b3_ref, hx_ref, u3_ref):
    x = x_ref[...]
    xb = x.astype(_BF16)
    hx = jnp.dot(xb, w1_ref[...].astype(_BF16), preferred_element_type=_F32)
    hx_ref[...] = hx.astype(_BF16)
    u3_ref[...] = jnp.dot(x, w3_ref[...], preferred_element_type=_F32) + b3_ref[...]


def _k1(x, W1, W3, b3r):
    return pl.pallas_call(
        _k1_body,
        out_shape=[jax.ShapeDtypeStruct((N, H), _BF16),
                   jax.ShapeDtypeStruct((N, H), _F32)],
    )(x, W1, W3, b3r)


def _k2_body(adj_ref, hx_ref, h1_ref, s_ref, ct_ref):
    i = pl.program_id(0)
    a = adj_ref[...]
    h1 = jnp.dot(a.astype(_BF16), hx_ref[...], preferred_element_type=_F32)
    h1_ref[...] = h1.astype(_BF16)
    rowg = i * BM + jax.lax.broadcasted_iota(jnp.int32, (BM, N), 0)
    colg = jax.lax.broadcasted_iota(jnp.int32, (BM, N), 1)
    am = jnp.where(colg > rowg, a, 0.0)
    s_ref[...] = jnp.sum(am, axis=1, keepdims=True)

    @pl.when(i == 0)
    def _():
        ct_ref[...] = jnp.zeros_like(ct_ref)

    ct_ref[...] += jnp.sum(am, axis=0, keepdims=True)


def _k2(adj, hx):
    return pl.pallas_call(
        _k2_body,
        grid=(N // BM,),
        in_specs=[pl.BlockSpec((BM, N), lambda i: (i, 0)),
                  pl.BlockSpec((N, H), lambda i: (0, 0))],
        out_specs=[pl.BlockSpec((BM, H), lambda i: (i, 0)),
                   pl.BlockSpec((BM, 1), lambda i: (i, 0)),
                   pl.BlockSpec((1, N), lambda i: (0, 0))],
        out_shape=[jax.ShapeDtypeStruct((N, H), _BF16),
                   jax.ShapeDtypeStruct((N, 1), _F32),
                   jax.ShapeDtypeStruct((1, N), _F32)],
        compiler_params=_ARB,
    )(adj, hx)


def _k3_body(adj_ref, h1_ref, w2_ref, z_ref, lii_ref, st_ref, g1):
    i = pl.program_id(0)

    @pl.when(i == 0)
    def _():
        g1[...] = jnp.dot(h1_ref[...], w2_ref[...].astype(_BF16),
                          preferred_element_type=_F32).astype(_BF16)

    z = jnp.maximum(jnp.dot(adj_ref[...].astype(_BF16), g1[...],
                            preferred_element_type=_F32), 0.0)
    z_ref[...] = z
    lii_ref[...] = jnp.sum(z * z, axis=1, keepdims=True)

    @pl.when(i == 0)
    def _():
        st_ref[...] = jnp.zeros_like(st_ref)

    st_ref[...] += jnp.sum(z, axis=0, keepdims=True)


def _k3(adj, h1, W2):
    return pl.pallas_call(
        _k3_body,
        grid=(N // BM,),
        in_specs=[pl.BlockSpec((BM, N), lambda i: (i, 0)),
                  pl.BlockSpec((N, H), lambda i: (0, 0)),
                  pl.BlockSpec((H, H), lambda i: (0, 0))],
        out_specs=[pl.BlockSpec((BM, H), lambda i: (i, 0)),
                   pl.BlockSpec((BM, 1), lambda i: (i, 0)),
                   pl.BlockSpec((1, H), lambda i: (0, 0))],
        out_shape=[jax.ShapeDtypeStruct((N, H), _F32),
                   jax.ShapeDtypeStruct((N, 1), _F32),
                   jax.ShapeDtypeStruct((1, H), _F32)],
        scratch_shapes=[pltpu.VMEM((N, H), _BF16)],
        compiler_params=_ARB,
    )(adj, h1, W2)


def _k5_body(lii_ref, s_col_ref, s_ref, c_ref, z_ref, zt_ref, u3_ref,
             d_ref, du3_ref, t3s_ref, liia_ref):
    lii = lii_ref[...]
    am = ALPHA / jnp.max(lii)
    zdots = jnp.sum(z_ref[...] * s_col_ref[...], axis=1, keepdims=True)
    rowsum = 1.0 + am * (zdots - lii) \
        + (1.0 - ALPHA) * (s_ref[...] + c_ref[...])
    d = jax.lax.rsqrt(rowsum)
    d_ref[...] = d
    du3 = d * u3_ref[...]
    du3_ref[...] = du3
    t3s_ref[...] = am * jnp.dot(zt_ref[...], du3, preferred_element_type=_F32)
    liia_ref[...] = am * lii


def _k5(lii, s_col, s, ccol, z, zt, u3):
    return pl.pallas_call(
        _k5_body,
        out_shape=[jax.ShapeDtypeStruct((N, 1), _F32),
                   jax.ShapeDtypeStruct((N, H), _F32),
                   jax.ShapeDtypeStruct((H, H), _F32),
                   jax.ShapeDtypeStruct((N, 1), _F32)],
    )(lii, s_col, s, ccol, z, zt, u3)


def _k6_body(rc_ref, adj_ref, dvc_ref, dvtr_ref, o1_ref, o2t_ref):
    t = pl.program_id(0)
    r = rc_ref[0, t]
    c = rc_ref[1, t]
    a = adj_ref[...].astype(_BF16)
    rowg = r * BT + jax.lax.broadcasted_iota(jnp.int32, (BT, BT), 0)
    colg = c * BT + jax.lax.broadcasted_iota(jnp.int32, (BT, BT), 1)
    am = jnp.where(colg > rowg, a, jnp.zeros_like(a))

    @pl.when(t == 0)
    def _():
        o1_ref[...] = jnp.zeros_like(o1_ref)
        o2t_ref[...] = jnp.zeros_like(o2t_ref)

    o1_ref[r] += jnp.dot(am, dvc_ref[...], preferred_element_type=_F32)
    o2t_ref[c] += jnp.dot(dvtr_ref[...], am, preferred_element_type=_F32)


def _k6(rc, adj, dvb, dvtb):
    grid_spec = pltpu.PrefetchScalarGridSpec(
        num_scalar_prefetch=1,
        grid=(NT,),
        in_specs=[pl.BlockSpec((BT, BT), lambda t, rc: (rc[0, t], rc[1, t])),
                  pl.BlockSpec((BT, H), lambda t, rc: (rc[1, t], 0)),
                  pl.BlockSpec((H, BT), lambda t, rc: (0, rc[0, t]))],
        out_specs=[pl.BlockSpec((NBT, BT, H), lambda t, rc: (0, 0, 0)),
                   pl.BlockSpec((NBT, H, BT), lambda t, rc: (0, 0, 0))],
    )
    return pl.pallas_call(
        _k6_body,
        grid_spec=grid_spec,
        out_shape=[jax.ShapeDtypeStruct((NBT, BT, H), _F32),
                   jax.ShapeDtypeStruct((NBT, H, BT), _F32)],
        compiler_params=_ARB,
    )(rc, adj, dvb, dvtb)


def _k6b_body(o1_ref, o2_ref, du3_ref, z_ref, t3s_ref, liia_ref, d_ref,
              w4_ref, b4_ref, zt_ref, lii_ref, dv2_ref, t4s_ref):
    du3 = du3_ref[...]
    core = jnp.dot(z_ref[...], t3s_ref[...], preferred_element_type=_F32) \
        - liia_ref[...] * du3 \
        + (1.0 - ALPHA) * (o1_ref[...] + o2_ref[...]) + du3
    h1c = jnp.maximum(d_ref[...] * core, 0.0)
    v2 = jnp.dot(h1c, w4_ref[...], preferred_element_type=_F32) + b4_ref[...]
    dv2 = d_ref[...] * v2
    dv2_ref[...] = dv2
    am = ALPHA / jnp.max(lii_ref[...])
    t4s_ref[...] = am * jnp.dot(zt_ref[...], dv2, preferred_element_type=_F32)


def _k6b(o1, o2, du3, z, t3s, liia, d, W4p, b4p, zt, lii):
    return pl.pallas_call(
        _k6b_body,
        out_shape=[jax.ShapeDtypeStruct((N, H), _F32),
                   jax.ShapeDtypeStruct((H, H), _F32)],
    )(o1, o2, du3, z, t3s, liia, d, W4p, b4p, zt, lii)


def _k7b_body(o1_ref, o2_ref, dv2_ref, z_ref, t4s_ref, liia_ref, d_ref,
              out_ref):
    dv2 = dv2_ref[...]
    core = jnp.dot(z_ref[...], t4s_ref[...], preferred_element_type=_F32) \
        - liia_ref[...] * dv2 \
        + (1.0 - ALPHA) * (o1_ref[...] + o2_ref[...]) + dv2
    out_ref[...] = d_ref[...] * core


def _k7b(o1, o2, dv2, z, t4s, liia, d):
    return pl.pallas_call(
        _k7b_body,
        out_shape=jax.ShapeDtypeStruct((N, H), _F32),
    )(o1, o2, dv2, z, t4s, liia, d)


def kernel(adj, x, W1, W2, W3, b3, W4, b4):
    nclass = W4.shape[1]
    b3r = b3.reshape(1, H).astype(_F32)
    W4p = jnp.zeros((H, H), _F32).at[:, :nclass].set(W4)
    b4p = jnp.zeros((1, H), _F32).at[0, :nclass].set(b4)
    rc = jnp.asarray(_UPPER)

    hx, u3 = _k1(x, W1, W3, b3r)
    h1, s, cT = _k2(adj, hx)
    z, lii, s_col = _k3(adj, h1, W2)
    zt = z.T
    ccol = cT.T
    d, du3, t3s, liia = _k5(lii, s_col, s, ccol, z, zt, u3)
    du3b = du3.astype(_BF16)
    du3tb = du3b.T
    o1a, o2ta = _k6(rc, adj, du3b, du3tb)
    o1 = o1a.reshape(N, H)
    o2 = jnp.transpose(o2ta, (0, 2, 1)).reshape(N, H)
    dv2, t4s = _k6b(o1, o2, du3, z, t3s, liia, d, W4p, b4p, zt, lii)
    dv2b = dv2.astype(_BF16)
    dv2tb = dv2b.T
    o1b, o2tb = _k6(rc, adj, dv2b, dv2tb)
    o1f = o1b.reshape(N, H)
    o2f = jnp.transpose(o2tb, (0, 2, 1)).reshape(N, H)
    outf = _k7b(o1f, o2f, dv2, z, t4s, liia, d)
    return outf[:, :nclass]


# speedup vs baseline: 2.0056x; 1.1206x over previous
import numpy as np
import jax
import jax.numpy as jnp
from jax.experimental import pallas as pl
from jax.experimental.pallas import tpu as pltpu

N = 4096
F = 256
H = 128
ALPHA = 0.8
BM = 256
BT = 512
NBT = N // BT
_UPPER = np.array([(r, c) for r in range(NBT) for c in range(r, NBT)],
                  dtype=np.int32).T.copy()
NT = _UPPER.shape[1]

_ARB = pltpu.CompilerParams(dimension_semantics=("arbitrary",))
_F32 = jnp.float32
_BF16 = jnp.bfloat16
_TLHS = (((0,), (0,)), ((), ()))


def _k1_body(x_ref, w1_ref, w3_ref, b3_ref, hx_ref, u3_ref):
    x = x_ref[...]
    xb = x.astype(_BF16)
    hx = jnp.dot(xb, w1_ref[...].astype(_BF16), preferred_element_type=_F32)
    hx_ref[...] = hx.astype(_BF16)
    u3_ref[...] = jnp.dot(x, w3_ref[...], preferred_element_type=_F32) + b3_ref[...]


def _k1(x, W1, W3, b3r):
    return pl.pallas_call(
        _k1_body,
        out_shape=[jax.ShapeDtypeStruct((N, H), _BF16),
                   jax.ShapeDtypeStruct((N, H), _F32)],
    )(x, W1, W3, b3r)


def _k2_body(adj_ref, hx_ref, h1_ref, s_ref, ct_ref):
    i = pl.program_id(0)
    a = adj_ref[...]
    h1 = jnp.dot(a.astype(_BF16), hx_ref[...], preferred_element_type=_F32)
    h1_ref[...] = h1.astype(_BF16)
    rowg = i * BM + jax.lax.broadcasted_iota(jnp.int32, (BM, N), 0)
    colg = jax.lax.broadcasted_iota(jnp.int32, (BM, N), 1)
    am = jnp.where(colg > rowg, a, 0.0)
    s_ref[...] = jnp.sum(am, axis=1, keepdims=True)

    @pl.when(i == 0)
    def _():
        ct_ref[...] = jnp.zeros_like(ct_ref)

    ct_ref[...] += jnp.sum(am, axis=0, keepdims=True)


def _k2(adj, hx):
    return pl.pallas_call(
        _k2_body,
        grid=(N // BM,),
        in_specs=[pl.BlockSpec((BM, N), lambda i: (i, 0)),
                  pl.BlockSpec((N, H), lambda i: (0, 0))],
        out_specs=[pl.BlockSpec((BM, H), lambda i: (i, 0)),
                   pl.BlockSpec((BM, 1), lambda i: (i, 0)),
                   pl.BlockSpec((1, N), lambda i: (0, 0))],
        out_shape=[jax.ShapeDtypeStruct((N, H), _BF16),
                   jax.ShapeDtypeStruct((N, 1), _F32),
                   jax.ShapeDtypeStruct((1, N), _F32)],
        compiler_params=_ARB,
    )(adj, hx)


def _k3_body(adj_ref, h1_ref, w2_ref, z_ref, lii_ref, st_ref, g1):
    i = pl.program_id(0)

    @pl.when(i == 0)
    def _():
        g1[...] = jnp.dot(h1_ref[...], w2_ref[...].astype(_BF16),
                          preferred_element_type=_F32).astype(_BF16)

    z = jnp.maximum(jnp.dot(adj_ref[...].astype(_BF16), g1[...],
                            preferred_element_type=_F32), 0.0)
    z_ref[...] = z
    lii_ref[...] = jnp.sum(z * z, axis=1, keepdims=True)

    @pl.when(i == 0)
    def _():
        st_ref[...] = jnp.zeros_like(st_ref)

    st_ref[...] += jnp.sum(z, axis=0, keepdims=True)


def _k3(adj, h1, W2):
    return pl.pallas_call(
        _k3_body,
        grid=(N // BM,),
        in_specs=[pl.BlockSpec((BM, N), lambda i: (i, 0)),
                  pl.BlockSpec((N, H), lambda i: (0, 0)),
                  pl.BlockSpec((H, H), lambda i: (0, 0))],
        out_specs=[pl.BlockSpec((BM, H), lambda i: (i, 0)),
                   pl.BlockSpec((BM, 1), lambda i: (i, 0)),
                   pl.BlockSpec((1, H), lambda i: (0, 0))],
        out_shape=[jax.ShapeDtypeStruct((N, H), _F32),
                   jax.ShapeDtypeStruct((N, 1), _F32),
                   jax.ShapeDtypeStruct((1, H), _F32)],
        scratch_shapes=[pltpu.VMEM((N, H), _BF16)],
        compiler_params=_ARB,
    )(adj, h1, W2)


def _k5_body(lii_ref, s_col_ref, s_ref, c_ref, z_ref, u3_ref,
             d_ref, du3_ref, du3b_ref, t3s_ref, liia_ref):
    lii = lii_ref[...]
    am = ALPHA / jnp.max(lii)
    zdots = jnp.sum(z_ref[...] * s_col_ref[...], axis=1, keepdims=True)
    rowsum = 1.0 + am * (zdots - lii) \
        + (1.0 - ALPHA) * (s_ref[...] + c_ref[...])
    d = jax.lax.rsqrt(rowsum)
    d_ref[...] = d
    du3 = d * u3_ref[...]
    du3_ref[...] = du3
    du3b_ref[...] = du3.astype(_BF16)
    t3s_ref[...] = am * jax.lax.dot_general(
        z_ref[...], du3, _TLHS, preferred_element_type=_F32)
    liia_ref[...] = am * lii


def _k5(lii, s_col, s, ccol, z, u3):
    return pl.pallas_call(
        _k5_body,
        out_shape=[jax.ShapeDtypeStruct((N, 1), _F32),
                   jax.ShapeDtypeStruct((N, H), _F32),
                   jax.ShapeDtypeStruct((N, H), _BF16),
                   jax.ShapeDtypeStruct((H, H), _F32),
                   jax.ShapeDtypeStruct((N, 1), _F32)],
    )(lii, s_col, s, ccol, z, u3)


def _accumulate_tile(rc_ref, adj_ref, dvc_ref, dvr_ref, maskc_ref, o1, o2):
    t = pl.program_id(0)
    r = rc_ref[0, t]
    c = rc_ref[1, t]
    a = adj_ref[...].astype(_BF16)
    am = jnp.where(r == c, a * maskc_ref[...], a)

    @pl.when(t == 0)
    def _():
        o1[...] = jnp.zeros_like(o1)
        o2[...] = jnp.zeros_like(o2)

    o1[r] += jnp.dot(am, dvc_ref[...], preferred_element_type=_F32)
    o2[c] += jax.lax.dot_general(dvr_ref[...], am, _TLHS,
                                 preferred_element_type=_F32)


def _core(o1, o2, z_ref, ts_ref, liia_ref, dv_ref):
    dv = dv_ref[...]
    o2rows = jnp.concatenate(
        [jnp.transpose(o2[cb]) for cb in range(NBT)], axis=0)
    return jnp.dot(z_ref[...], ts_ref[...], preferred_element_type=_F32) \
        - liia_ref[...] * dv \
        + (1.0 - ALPHA) * (o1[...].reshape(N, H) + o2rows) \
        + dv


def _k6_body(rc_ref, adj_ref, dvc_ref, dvr_ref, maskc_ref,
             z_ref, du3_ref, t3s_ref, liia_ref, d_ref, w4_ref, b4_ref,
             lii_ref, dv2_ref, dv2b_ref, t4s_ref, o1, o2):
    _accumulate_tile(rc_ref, adj_ref, dvc_ref, dvr_ref, maskc_ref, o1, o2)

    @pl.when(pl.program_id(0) == NT - 1)
    def _():
        core = _core(o1, o2, z_ref, t3s_ref, liia_ref, du3_ref)
        h1c = jnp.maximum(d_ref[...] * core, 0.0)
        v2 = jnp.dot(h1c, w4_ref[...], preferred_element_type=_F32) + b4_ref[...]
        dv2 = d_ref[...] * v2
        dv2_ref[...] = dv2
        dv2b_ref[...] = dv2.astype(_BF16)
        am = ALPHA / jnp.max(lii_ref[...])
        t4s_ref[...] = am * jax.lax.dot_general(
            z_ref[...], dv2, _TLHS, preferred_element_type=_F32)


def _k7_body(rc_ref, adj_ref, dvc_ref, dvr_ref, maskc_ref,
             z_ref, dv2_ref, t4s_ref, liia_ref, d_ref, out_ref, o1, o2):
    _accumulate_tile(rc_ref, adj_ref, dvc_ref, dvr_ref, maskc_ref, o1, o2)

    @pl.when(pl.program_id(0) == NT - 1)
    def _():
        core = _core(o1, o2, z_ref, t4s_ref, liia_ref, dv2_ref)
        out_ref[...] = d_ref[...] * core


def _tile_specs():
    return [pl.BlockSpec((BT, BT), lambda t, rc: (rc[0, t], rc[1, t])),
            pl.BlockSpec((BT, H), lambda t, rc: (rc[1, t], 0)),
            pl.BlockSpec((BT, H), lambda t, rc: (rc[0, t], 0)),
            pl.BlockSpec((BT, BT), lambda t, rc: (0, 0))]


def _const2(shape):
    return pl.BlockSpec(shape, lambda t, rc: (0, 0))


_ACC_SCRATCH = [pltpu.VMEM((NBT, BT, H), _F32), pltpu.VMEM((NBT, H, BT), _F32)]


def _k6(rc, adj, du3b, maskc, z, du3, t3s, liia, d, W4p, b4p, lii):
    grid_spec = pltpu.PrefetchScalarGridSpec(
        num_scalar_prefetch=1,
        grid=(NT,),
        in_specs=_tile_specs() + [
            _const2((N, H)),
            _const2((N, H)),
            _const2((H, H)),
            _const2((N, 1)),
            _const2((N, 1)),
            _const2((H, H)),
            _const2((1, H)),
            _const2((N, 1)),
        ],
        out_specs=[_const2((N, H)), _const2((N, H)), _const2((H, H))],
        scratch_shapes=_ACC_SCRATCH,
    )
    return pl.pallas_call(
        _k6_body,
        grid_spec=grid_spec,
        out_shape=[jax.ShapeDtypeStruct((N, H), _F32),
                   jax.ShapeDtypeStruct((N, H), _BF16),
                   jax.ShapeDtypeStruct((H, H), _F32)],
        compiler_params=_ARB,
    )(rc, adj, du3b, du3b, maskc, z, du3, t3s, liia, d, W4p, b4p, lii)


def _k7(rc, adj, dv2b, maskc, z, dv2, t4s, liia, d):
    grid_spec = pltpu.PrefetchScalarGridSpec(
        num_scalar_prefetch=1,
        grid=(NT,),
        in_specs=_tile_specs() + [
            _const2((N, H)),
            _const2((N, H)),
            _const2((H, H)),
            _const2((N, 1)),
            _const2((N, 1)),
        ],
        out_specs=_const2((N, H)),
        scratch_shapes=_ACC_SCRATCH,
    )
    return pl.pallas_call(
        _k7_body,
        grid_spec=grid_spec,
        out_shape=jax.ShapeDtypeStruct((N, H), _F32),
        compiler_params=_ARB,
    )(rc, adj, dv2b, dv2b, maskc, z, dv2, t4s, liia, d)


def kernel(adj, x, W1, W2, W3, b3, W4, b4):
    nclass = W4.shape[1]
    b3r = b3.reshape(1, H).astype(_F32)
    W4p = jnp.zeros((H, H), _F32).at[:, :nclass].set(W4)
    b4p = jnp.zeros((1, H), _F32).at[0, :nclass].set(b4)
    rc = jnp.asarray(_UPPER)
    maskc = jnp.asarray(
        np.triu(np.ones((BT, BT), np.float32), 1).astype(np.float32),
        dtype=_BF16)

    hx, u3 = _k1(x, W1, W3, b3r)
    h1, s, cT = _k2(adj, hx)
    z, lii, s_col = _k3(adj, h1, W2)
    ccol = cT.T
    d, du3, du3b, t3s, liia = _k5(lii, s_col, s, ccol, z, u3)
    dv2, dv2b, t4s = _k6(rc, adj, du3b, maskc, z, du3, t3s, liia, d,
                         W4p, b4p, lii)
    outf = _k7(rc, adj, dv2b, maskc, z, dv2, t4s, liia, d)
    return outf[:, :nclass]


# speedup vs baseline: 2.0363x; 1.0153x over previous
import numpy as np
import jax
import jax.numpy as jnp
from jax.experimental import pallas as pl
from jax.experimental.pallas import tpu as pltpu

N = 4096
F = 256
H = 128
ALPHA = 0.8
BM = 256
BT = 512
NBT = N // BT
_UPPER = np.array([(r, c) for r in range(NBT) for c in range(r, NBT)],
                  dtype=np.int32).T.copy()
NT = _UPPER.shape[1]

_ARB = pltpu.CompilerParams(dimension_semantics=("arbitrary",))
_F32 = jnp.float32
_BF16 = jnp.bfloat16
_TLHS = (((0,), (0,)), ((), ()))


def _k1_body(x_ref, w1_ref, w3_ref, b3_ref, hx_ref, u3_ref):
    x = x_ref[...]
    xb = x.astype(_BF16)
    hx = jnp.dot(xb, w1_ref[...].astype(_BF16), preferred_element_type=_F32)
    hx_ref[...] = hx.astype(_BF16)
    u3_ref[...] = jnp.dot(x, w3_ref[...], preferred_element_type=_F32) + b3_ref[...]


def _k1(x, W1, W3, b3r):
    return pl.pallas_call(
        _k1_body,
        out_shape=[jax.ShapeDtypeStruct((N, H), _BF16),
                   jax.ShapeDtypeStruct((N, H), _F32)],
    )(x, W1, W3, b3r)


def _k2_body(adj_ref, hx_ref, h1_ref, s_ref, ct_ref, au_ref):
    i = pl.program_id(0)
    a = adj_ref[...]
    h1 = jnp.dot(a.astype(_BF16), hx_ref[...], preferred_element_type=_F32)
    h1_ref[...] = h1.astype(_BF16)
    rowg = i * BM + jax.lax.broadcasted_iota(jnp.int32, (BM, N), 0)
    colg = jax.lax.broadcasted_iota(jnp.int32, (BM, N), 1)
    am = jnp.where(colg > rowg, a, 0.0)
    au_ref[...] = am.astype(_BF16)
    s_ref[...] = jnp.sum(am, axis=1, keepdims=True)

    @pl.when(i == 0)
    def _():
        ct_ref[...] = jnp.zeros_like(ct_ref)

    ct_ref[...] += jnp.sum(am, axis=0, keepdims=True)


def _k2(adj, hx):
    return pl.pallas_call(
        _k2_body,
        grid=(N // BM,),
        in_specs=[pl.BlockSpec((BM, N), lambda i: (i, 0)),
                  pl.BlockSpec((N, H), lambda i: (0, 0))],
        out_specs=[pl.BlockSpec((BM, H), lambda i: (i, 0)),
                   pl.BlockSpec((BM, 1), lambda i: (i, 0)),
                   pl.BlockSpec((1, N), lambda i: (0, 0)),
                   pl.BlockSpec((BM, N), lambda i: (i, 0))],
        out_shape=[jax.ShapeDtypeStruct((N, H), _BF16),
                   jax.ShapeDtypeStruct((N, 1), _F32),
                   jax.ShapeDtypeStruct((1, N), _F32),
                   jax.ShapeDtypeStruct((N, N), _BF16)],
        compiler_params=_ARB,
    )(adj, hx)


def _k3_body(adj_ref, h1_ref, w2_ref, z_ref, lii_ref, st_ref, g1):
    i = pl.program_id(0)

    @pl.when(i == 0)
    def _():
        g1[...] = jnp.dot(h1_ref[...], w2_ref[...].astype(_BF16),
                          preferred_element_type=_F32).astype(_BF16)

    z = jnp.maximum(jnp.dot(adj_ref[...].astype(_BF16), g1[...],
                            preferred_element_type=_F32), 0.0)
    z_ref[...] = z
    lii_ref[...] = jnp.sum(z * z, axis=1, keepdims=True)

    @pl.when(i == 0)
    def _():
        st_ref[...] = jnp.zeros_like(st_ref)

    st_ref[...] += jnp.sum(z, axis=0, keepdims=True)


def _k3(adj, h1, W2):
    return pl.pallas_call(
        _k3_body,
        grid=(N // BM,),
        in_specs=[pl.BlockSpec((BM, N), lambda i: (i, 0)),
                  pl.BlockSpec((N, H), lambda i: (0, 0)),
                  pl.BlockSpec((H, H), lambda i: (0, 0))],
        out_specs=[pl.BlockSpec((BM, H), lambda i: (i, 0)),
                   pl.BlockSpec((BM, 1), lambda i: (i, 0)),
                   pl.BlockSpec((1, H), lambda i: (0, 0))],
        out_shape=[jax.ShapeDtypeStruct((N, H), _F32),
                   jax.ShapeDtypeStruct((N, 1), _F32),
                   jax.ShapeDtypeStruct((1, H), _F32)],
        scratch_shapes=[pltpu.VMEM((N, H), _BF16)],
        compiler_params=_ARB,
    )(adj, h1, W2)


def _k5_body(lii_ref, s_col_ref, s_ref, c_ref, z_ref, u3_ref,
             d_ref, du3_ref, du3b_ref, t3s_ref, liia_ref):
    lii = lii_ref[...]
    am = ALPHA / jnp.max(lii)
    zdots = jnp.sum(z_ref[...] * s_col_ref[...], axis=1, keepdims=True)
    rowsum = 1.0 + am * (zdots - lii) \
        + (1.0 - ALPHA) * (s_ref[...] + c_ref[...])
    d = jax.lax.rsqrt(rowsum)
    d_ref[...] = d
    du3 = d * u3_ref[...]
    du3_ref[...] = du3
    du3b_ref[...] = du3.astype(_BF16)
    t3s_ref[...] = am * jax.lax.dot_general(
        z_ref[...], du3, _TLHS, preferred_element_type=_F32)
    liia_ref[...] = am * lii


def _k5(lii, s_col, s, ccol, z, u3):
    return pl.pallas_call(
        _k5_body,
        out_shape=[jax.ShapeDtypeStruct((N, 1), _F32),
                   jax.ShapeDtypeStruct((N, H), _F32),
                   jax.ShapeDtypeStruct((N, H), _BF16),
                   jax.ShapeDtypeStruct((H, H), _F32),
                   jax.ShapeDtypeStruct((N, 1), _F32)],
    )(lii, s_col, s, ccol, z, u3)


def _accumulate_tile(rc_ref, adj_ref, dvc_ref, dvr_ref, o1, o2):
    t = pl.program_id(0)
    r = rc_ref[0, t]
    c = rc_ref[1, t]
    am = adj_ref[...]

    @pl.when(t == 0)
    def _():
        o1[...] = jnp.zeros_like(o1)
        o2[...] = jnp.zeros_like(o2)

    o1[r] += jnp.dot(am, dvc_ref[...], preferred_element_type=_F32)
    o2[c] += jax.lax.dot_general(dvr_ref[...], am, _TLHS,
                                 preferred_element_type=_F32)


def _core(o1, o2, z_ref, ts_ref, liia_ref, dv_ref):
    dv = dv_ref[...]
    o2rows = jnp.concatenate(
        [jnp.transpose(o2[cb]) for cb in range(NBT)], axis=0)
    return jnp.dot(z_ref[...], ts_ref[...], preferred_element_type=_F32) \
        - liia_ref[...] * dv \
        + (1.0 - ALPHA) * (o1[...].reshape(N, H) + o2rows) \
        + dv


def _k6_body(rc_ref, adj_ref, dvc_ref, dvr_ref,
             z_ref, du3_ref, t3s_ref, liia_ref, d_ref, w4_ref, b4_ref,
             lii_ref, dv2_ref, dv2b_ref, t4s_ref, o1, o2):
    _accumulate_tile(rc_ref, adj_ref, dvc_ref, dvr_ref, o1, o2)

    @pl.when(pl.program_id(0) == NT - 1)
    def _():
        core = _core(o1, o2, z_ref, t3s_ref, liia_ref, du3_ref)
        h1c = jnp.maximum(d_ref[...] * core, 0.0)
        v2 = jnp.dot(h1c, w4_ref[...], preferred_element_type=_F32) + b4_ref[...]
        dv2 = d_ref[...] * v2
        dv2_ref[...] = dv2
        dv2b_ref[...] = dv2.astype(_BF16)
        am = ALPHA / jnp.max(lii_ref[...])
        t4s_ref[...] = am * jax.lax.dot_general(
            z_ref[...], dv2, _TLHS, preferred_element_type=_F32)


def _k7_body(rc_ref, adj_ref, dvc_ref, dvr_ref,
             z_ref, dv2_ref, t4s_ref, liia_ref, d_ref, out_ref, o1, o2):
    _accumulate_tile(rc_ref, adj_ref, dvc_ref, dvr_ref, o1, o2)

    @pl.when(pl.program_id(0) == NT - 1)
    def _():
        core = _core(o1, o2, z_ref, t4s_ref, liia_ref, dv2_ref)
        out_ref[...] = d_ref[...] * core


def _tile_specs():
    return [pl.BlockSpec((BT, BT), lambda t, rc: (rc[0, t], rc[1, t])),
            pl.BlockSpec((BT, H), lambda t, rc: (rc[1, t], 0)),
            pl.BlockSpec((BT, H), lambda t, rc: (rc[0, t], 0))]


def _const2(shape):
    return pl.BlockSpec(shape, lambda t, rc: (0, 0))


_ACC_SCRATCH = [pltpu.VMEM((NBT, BT, H), _F32), pltpu.VMEM((NBT, H, BT), _F32)]


def _k6(rc, adjU, du3b, z, du3, t3s, liia, d, W4p, b4p, lii):
    grid_spec = pltpu.PrefetchScalarGridSpec(
        num_scalar_prefetch=1,
        grid=(NT,),
        in_specs=_tile_specs() + [
            _const2((N, H)),
            _const2((N, H)),
            _const2((H, H)),
            _const2((N, 1)),
            _const2((N, 1)),
            _const2((H, H)),
            _const2((1, H)),
            _const2((N, 1)),
        ],
        out_specs=[_const2((N, H)), _const2((N, H)), _const2((H, H))],
        scratch_shapes=_ACC_SCRATCH,
    )
    return pl.pallas_call(
        _k6_body,
        grid_spec=grid_spec,
        out_shape=[jax.ShapeDtypeStruct((N, H), _F32),
                   jax.ShapeDtypeStruct((N, H), _BF16),
                   jax.ShapeDtypeStruct((H, H), _F32)],
        compiler_params=_ARB,
    )(rc, adjU, du3b, du3b, z, du3, t3s, liia, d, W4p, b4p, lii)


def _k7(rc, adjU, dv2b, z, dv2, t4s, liia, d):
    grid_spec = pltpu.PrefetchScalarGridSpec(
        num_scalar_prefetch=1,
        grid=(NT,),
        in_specs=_tile_specs() + [
            _const2((N, H)),
            _const2((N, H)),
            _const2((H, H)),
            _const2((N, 1)),
            _const2((N, 1)),
        ],
        out_specs=_const2((N, H)),
        scratch_shapes=_ACC_SCRATCH,
    )
    return pl.pallas_call(
        _k7_body,
        grid_spec=grid_spec,
        out_shape=jax.ShapeDtypeStruct((N, H), _F32),
        compiler_params=_ARB,
    )(rc, adjU, dv2b, dv2b, z, dv2, t4s, liia, d)


def kernel(adj, x, W1, W2, W3, b3, W4, b4):
    nclass = W4.shape[1]
    b3r = b3.reshape(1, H).astype(_F32)
    W4p = jnp.zeros((H, H), _F32).at[:, :nclass].set(W4)
    b4p = jnp.zeros((1, H), _F32).at[0, :nclass].set(b4)
    rc = jnp.asarray(_UPPER)

    hx, u3 = _k1(x, W1, W3, b3r)
    h1, s, cT, adjU = _k2(adj, hx)
    z, lii, s_col = _k3(adj, h1, W2)
    ccol = cT.T
    d, du3, du3b, t3s, liia = _k5(lii, s_col, s, ccol, z, u3)
    dv2, dv2b, t4s = _k6(rc, adjU, du3b, z, du3, t3s, liia, d,
                         W4p, b4p, lii)
    outf = _k7(rc, adjU, dv2b, z, dv2, t4s, liia, d)
    return outf[:, :nclass]


# speedup vs baseline: 2.1773x; 1.0692x over previous
import numpy as np
import jax
import jax.numpy as jnp
from jax.experimental import pallas as pl
from jax.experimental.pallas import tpu as pltpu

N = 4096
F = 256
H = 128
ALPHA = 0.8
BM = 256
BT = 512
NBT = N // BT

_ARB = pltpu.CompilerParams(dimension_semantics=("arbitrary",))
_F32 = jnp.float32
_BF16 = jnp.bfloat16


def _k1_body(x_ref, w1_ref, w3_ref, b3_ref, hx_ref, u3_ref):
    x = x_ref[...]
    xb = x.astype(_BF16)
    hx = jnp.dot(xb, w1_ref[...].astype(_BF16), preferred_element_type=_F32)
    hx_ref[...] = hx.astype(_BF16)
    u3_ref[...] = jnp.dot(x, w3_ref[...], preferred_element_type=_F32) + b3_ref[...]


def _k1(x, W1, W3, b3r):
    return pl.pallas_call(
        _k1_body,
        out_shape=[jax.ShapeDtypeStruct((N, H), _BF16),
                   jax.ShapeDtypeStruct((N, H), _F32)],
    )(x, W1, W3, b3r)


def _k2_body(adj_ref, hx_ref, h1_ref, s_ref, ct_ref, au_ref):
    i = pl.program_id(0)
    a = adj_ref[...]
    h1 = jnp.dot(a.astype(_BF16), hx_ref[...], preferred_element_type=_F32)
    h1_ref[...] = h1.astype(_BF16)
    rowg = i * BM + jax.lax.broadcasted_iota(jnp.int32, (BM, N), 0)
    colg = jax.lax.broadcasted_iota(jnp.int32, (BM, N), 1)
    am = jnp.where(colg > rowg, a, 0.0)
    au_ref[...] = am.astype(_BF16)
    s_ref[...] = jnp.sum(am, axis=1, keepdims=True)

    @pl.when(i == 0)
    def _():
        ct_ref[...] = jnp.zeros_like(ct_ref)

    ct_ref[...] += jnp.sum(am, axis=0, keepdims=True)


def _k2(adj, hx):
    return pl.pallas_call(
        _k2_body,
        grid=(N // BM,),
        in_specs=[pl.BlockSpec((BM, N), lambda i: (i, 0)),
                  pl.BlockSpec((N, H), lambda i: (0, 0))],
        out_specs=[pl.BlockSpec((BM, H), lambda i: (i, 0)),
                   pl.BlockSpec((BM, 1), lambda i: (i, 0)),
                   pl.BlockSpec((1, N), lambda i: (0, 0)),
                   pl.BlockSpec((BM, N), lambda i: (i, 0))],
        out_shape=[jax.ShapeDtypeStruct((N, H), _BF16),
                   jax.ShapeDtypeStruct((N, 1), _F32),
                   jax.ShapeDtypeStruct((1, N), _F32),
                   jax.ShapeDtypeStruct((N, N), _BF16)],
        compiler_params=_ARB,
    )(adj, hx)


def _k3_body(adj_ref, h1_ref, w2_ref, z_ref, lii_ref, st_ref, g1):
    i = pl.program_id(0)

    @pl.when(i == 0)
    def _():
        g1[...] = jnp.dot(h1_ref[...], w2_ref[...].astype(_BF16),
                          preferred_element_type=_F32).astype(_BF16)

    z = jnp.maximum(jnp.dot(adj_ref[...].astype(_BF16), g1[...],
                            preferred_element_type=_F32), 0.0)
    z_ref[...] = z
    lii_ref[...] = jnp.sum(z * z, axis=1, keepdims=True)

    @pl.when(i == 0)
    def _():
        st_ref[...] = jnp.zeros_like(st_ref)

    st_ref[...] += jnp.sum(z, axis=0, keepdims=True)


def _k3(adj, h1, W2):
    return pl.pallas_call(
        _k3_body,
        grid=(N // BM,),
        in_specs=[pl.BlockSpec((BM, N), lambda i: (i, 0)),
                  pl.BlockSpec((N, H), lambda i: (0, 0)),
                  pl.BlockSpec((H, H), lambda i: (0, 0))],
        out_specs=[pl.BlockSpec((BM, H), lambda i: (i, 0)),
                   pl.BlockSpec((BM, 1), lambda i: (i, 0)),
                   pl.BlockSpec((1, H), lambda i: (0, 0))],
        out_shape=[jax.ShapeDtypeStruct((N, H), _F32),
                   jax.ShapeDtypeStruct((N, 1), _F32),
                   jax.ShapeDtypeStruct((1, H), _F32)],
        scratch_shapes=[pltpu.VMEM((N, H), _BF16)],
        compiler_params=_ARB,
    )(adj, h1, W2)


def _k5_body(lii_ref, s_col_ref, s_ref, c_ref, z_ref, u3_ref,
             d_ref, du3_ref, du3b_ref, du3tb_ref, t3s_ref, liia_ref):
    lii = lii_ref[...]
    am = ALPHA / jnp.max(lii)
    zdots = jnp.sum(z_ref[...] * s_col_ref[...], axis=1, keepdims=True)
    rowsum = 1.0 + am * (zdots - lii) \
        + (1.0 - ALPHA) * (s_ref[...] + c_ref[...])
    d = jax.lax.rsqrt(rowsum)
    d_ref[...] = d
    du3 = d * u3_ref[...]
    du3_ref[...] = du3
    du3b_ref[...] = du3.astype(_BF16)
    du3tb_ref[...] = jnp.transpose(du3).astype(_BF16)
    t3s_ref[...] = am * jax.lax.dot_general(
        z_ref[...], du3, (((0,), (0,)), ((), ())), preferred_element_type=_F32)
    liia_ref[...] = am * lii


def _k5(lii, s_col, s, ccol, z, u3):
    return pl.pallas_call(
        _k5_body,
        out_shape=[jax.ShapeDtypeStruct((N, 1), _F32),
                   jax.ShapeDtypeStruct((N, H), _F32),
                   jax.ShapeDtypeStruct((N, H), _BF16),
                   jax.ShapeDtypeStruct((H, N), _BF16),
                   jax.ShapeDtypeStruct((H, H), _F32),
                   jax.ShapeDtypeStruct((N, 1), _F32)],
    )(lii, s_col, s, ccol, z, u3)


def _panel_step(aurow_ref, aucol_ref, dvb_ref, dvtb_ref, o1, o2t):
    r = pl.program_id(0)
    o1[r] = jnp.dot(aurow_ref[...], dvb_ref[...], preferred_element_type=_F32)
    o2t[r] = jnp.dot(dvtb_ref[...], aucol_ref[...], preferred_element_type=_F32)


def _core(o1, o2t, z_ref, ts_ref, liia_ref, dv_ref):
    dv = dv_ref[...]
    o2rows = jnp.concatenate(
        [jnp.transpose(o2t[cb]) for cb in range(NBT)], axis=0)
    return jnp.dot(z_ref[...], ts_ref[...], preferred_element_type=_F32) \
        - liia_ref[...] * dv \
        + (1.0 - ALPHA) * (o1[...].reshape(N, H) + o2rows) \
        + dv


def _k6_body(aurow_ref, aucol_ref, dvb_ref, dvtb_ref,
             z_ref, du3_ref, t3s_ref, liia_ref, d_ref, w4_ref, b4_ref,
             lii_ref, dv2_ref, dv2b_ref, dv2tb_ref, t4s_ref, o1, o2t):
    _panel_step(aurow_ref, aucol_ref, dvb_ref, dvtb_ref, o1, o2t)

    @pl.when(pl.program_id(0) == NBT - 1)
    def _():
        core = _core(o1, o2t, z_ref, t3s_ref, liia_ref, du3_ref)
        h1c = jnp.maximum(d_ref[...] * core, 0.0)
        v2 = jnp.dot(h1c, w4_ref[...], preferred_element_type=_F32) + b4_ref[...]
        dv2 = d_ref[...] * v2
        dv2_ref[...] = dv2
        dv2b = dv2.astype(_BF16)
        dv2b_ref[...] = dv2b
        dv2tb_ref[...] = jnp.transpose(dv2).astype(_BF16)
        amax = ALPHA / jnp.max(lii_ref[...])
        t4s_ref[...] = amax * jax.lax.dot_general(
            z_ref[...], dv2, (((0,), (0,)), ((), ())),
            preferred_element_type=_F32)


def _k7_body(aurow_ref, aucol_ref, dvb_ref, dvtb_ref,
             z_ref, dv2_ref, t4s_ref, liia_ref, d_ref, out_ref, o1, o2t):
    _panel_step(aurow_ref, aucol_ref, dvb_ref, dvtb_ref, o1, o2t)

    @pl.when(pl.program_id(0) == NBT - 1)
    def _():
        core = _core(o1, o2t, z_ref, t4s_ref, liia_ref, dv2_ref)
        out_ref[...] = d_ref[...] * core


def _panel_specs():
    return [pl.BlockSpec((BT, N), lambda r: (r, 0)),
            pl.BlockSpec((N, BT), lambda r: (0, r)),
            pl.BlockSpec((N, H), lambda r: (0, 0)),
            pl.BlockSpec((H, N), lambda r: (0, 0))]


def _const1(shape):
    return pl.BlockSpec(shape, lambda r: (0,) * len(shape))


_ACC_SCRATCH = [pltpu.VMEM((NBT, BT, H), _F32), pltpu.VMEM((NBT, H, BT), _F32)]


def _k6(adjU, du3b, du3tb, z, du3, t3s, liia, d, W4p, b4p, lii):
    return pl.pallas_call(
        _k6_body,
        grid=(NBT,),
        in_specs=_panel_specs() + [
            _const1((N, H)),
            _const1((N, H)),
            _const1((H, H)),
            _const1((N, 1)),
            _const1((N, 1)),
            _const1((H, H)),
            _const1((1, H)),
            _const1((N, 1)),
        ],
        out_specs=[_const1((N, H)), _const1((N, H)), _const1((H, N)),
                   _const1((H, H))],
        out_shape=[jax.ShapeDtypeStruct((N, H), _F32),
                   jax.ShapeDtypeStruct((N, H), _BF16),
                   jax.ShapeDtypeStruct((H, N), _BF16),
                   jax.ShapeDtypeStruct((H, H), _F32)],
        scratch_shapes=_ACC_SCRATCH,
        compiler_params=_ARB,
    )(adjU, adjU, du3b, du3tb, z, du3, t3s, liia, d, W4p, b4p, lii)


def _k7(adjU, dv2b, dv2tb, z, dv2, t4s, liia, d):
    return pl.pallas_call(
        _k7_body,
        grid=(NBT,),
        in_specs=_panel_specs() + [
            _const1((N, H)),
            _const1((N, H)),
            _const1((H, H)),
            _const1((N, 1)),
            _const1((N, 1)),
        ],
        out_specs=_const1((N, H)),
        out_shape=jax.ShapeDtypeStruct((N, H), _F32),
        scratch_shapes=_ACC_SCRATCH,
        compiler_params=_ARB,
    )(adjU, adjU, dv2b, dv2tb, z, dv2, t4s, liia, d)


def kernel(adj, x, W1, W2, W3, b3, W4, b4):
    nclass = W4.shape[1]
    b3r = b3.reshape(1, H).astype(_F32)
    W4p = jnp.zeros((H, H), _F32).at[:, :nclass].set(W4)
    b4p = jnp.zeros((1, H), _F32).at[0, :nclass].set(b4)

    hx, u3 = _k1(x, W1, W3, b3r)
    h1, s, cT, adjU = _k2(adj, hx)
    z, lii, s_col = _k3(adj, h1, W2)
    ccol = cT.T
    d, du3, du3b, du3tb, t3s, liia = _k5(lii, s_col, s, ccol, z, u3)
    dv2, dv2b, dv2tb, t4s = _k6(adjU, du3b, du3tb, z, du3, t3s, liia, d,
                                W4p, b4p, lii)
    outf = _k7(adjU, dv2b, dv2tb, z, dv2, t4s, liia, d)
    return outf[:, :nclass]


# speedup vs baseline: 2.4042x; 1.1042x over previous
import numpy as np
import jax
import jax.numpy as jnp
from jax.experimental import pallas as pl
from jax.experimental.pallas import tpu as pltpu

N = 4096
F = 256
H = 128
ALPHA = 0.8
BM = 256
BT = 512
NBT = N // BT
NB0 = N // BM
TOT = NB0 + 2 * NBT

_ARB = pltpu.CompilerParams(dimension_semantics=("arbitrary",))
_F32 = jnp.float32
_BF16 = jnp.bfloat16
_TLHS = (((0,), (0,)), ((), ()))


def _k1_body(x_ref, w1_ref, w3_ref, b3_ref, hx_ref, u3_ref):
    x = x_ref[...]
    xb = x.astype(_BF16)
    hx = jnp.dot(xb, w1_ref[...].astype(_BF16), preferred_element_type=_F32)
    hx_ref[...] = hx.astype(_BF16)
    u3_ref[...] = jnp.dot(x, w3_ref[...], preferred_element_type=_F32) + b3_ref[...]


def _k1(x, W1, W3, b3r):
    return pl.pallas_call(
        _k1_body,
        out_shape=[jax.ShapeDtypeStruct((N, H), _BF16),
                   jax.ShapeDtypeStruct((N, H), _F32)],
    )(x, W1, W3, b3r)


def _k2_body(adj_ref, hx_ref, h1_ref, s_ref, ct_ref, au_ref):
    i = pl.program_id(0)
    a = adj_ref[...]
    h1 = jnp.dot(a.astype(_BF16), hx_ref[...], preferred_element_type=_F32)
    h1_ref[...] = h1.astype(_BF16)
    rowg = i * BM + jax.lax.broadcasted_iota(jnp.int32, (BM, N), 0)
    colg = jax.lax.broadcasted_iota(jnp.int32, (BM, N), 1)
    am = jnp.where(colg > rowg, a, 0.0)
    au_ref[...] = am.astype(_BF16)
    s_ref[...] = jnp.sum(am, axis=1, keepdims=True)

    @pl.when(i == 0)
    def _():
        ct_ref[...] = jnp.zeros_like(ct_ref)

    ct_ref[...] += jnp.sum(am, axis=0, keepdims=True)


def _k2(adj, hx):
    return pl.pallas_call(
        _k2_body,
        grid=(NB0,),
        in_specs=[pl.BlockSpec((BM, N), lambda i: (i, 0)),
                  pl.BlockSpec((N, H), lambda i: (0, 0))],
        out_specs=[pl.BlockSpec((BM, H), lambda i: (i, 0)),
                   pl.BlockSpec((BM, 1), lambda i: (i, 0)),
                   pl.BlockSpec((1, N), lambda i: (0, 0)),
                   pl.BlockSpec((BM, N), lambda i: (i, 0))],
        out_shape=[jax.ShapeDtypeStruct((N, H), _BF16),
                   jax.ShapeDtypeStruct((N, 1), _F32),
                   jax.ShapeDtypeStruct((1, N), _F32),
                   jax.ShapeDtypeStruct((N, N), _BF16)],
        compiler_params=_ARB,
    )(adj, hx)


def _kbig_body(adj_ref, aurow_ref, aucol_ref, h1_ref, w2_ref, s_ref, c_ref,
               u3_ref, w4_ref, b4_ref, out_ref,
               g1, z_s, lii_s, sc_s, d_s, liia_s, dvf_s, dvb_s, dvtb_s,
               ts_s, o1, o2t):
    r = pl.program_id(0)

    @pl.when(r == 0)
    def _():
        g1[...] = jnp.dot(h1_ref[...], w2_ref[...].astype(_BF16),
                          preferred_element_type=_F32).astype(_BF16)
        sc_s[...] = jnp.zeros_like(sc_s)

    @pl.when(r < NB0)
    def _():
        zb = jnp.maximum(jnp.dot(adj_ref[...].astype(_BF16), g1[...],
                                 preferred_element_type=_F32), 0.0)
        z_s[pl.ds(r * BM, BM), :] = zb
        lii_s[pl.ds(r * BM, BM), :] = jnp.sum(zb * zb, axis=1, keepdims=True)
        sc_s[...] += jnp.sum(zb, axis=0, keepdims=True)

    @pl.when(r == NB0 - 1)
    def _():
        lii = lii_s[...]
        am = ALPHA / jnp.max(lii)
        z = z_s[...]
        zdots = jnp.sum(z * sc_s[...], axis=1, keepdims=True)
        rowsum = 1.0 + am * (zdots - lii) \
            + (1.0 - ALPHA) * (s_ref[...] + c_ref[...])
        d = jax.lax.rsqrt(rowsum)
        d_s[...] = d
        liia_s[...] = am * lii
        du3 = d * u3_ref[...]
        dvf_s[...] = du3
        dvb_s[...] = du3.astype(_BF16)
        dvtb_s[...] = jnp.transpose(du3).astype(_BF16)
        ts_s[...] = am * jax.lax.dot_general(z, du3, _TLHS,
                                             preferred_element_type=_F32)

    @pl.when(r >= NB0)
    def _():
        rr = (r - NB0) % NBT
        o1[rr] = jnp.dot(aurow_ref[...], dvb_s[...],
                         preferred_element_type=_F32)
        o2t[rr] = jnp.dot(dvtb_s[...], aucol_ref[...],
                          preferred_element_type=_F32)

    def core():
        dv = dvf_s[...]
        o2rows = jnp.concatenate(
            [jnp.transpose(o2t[cb]) for cb in range(NBT)], axis=0)
        return jnp.dot(z_s[...], ts_s[...], preferred_element_type=_F32) \
            - liia_s[...] * dv \
            + (1.0 - ALPHA) * (o1[...].reshape(N, H) + o2rows) \
            + dv

    @pl.when(r == NB0 + NBT - 1)
    def _():
        h1c = jnp.maximum(d_s[...] * core(), 0.0)
        v2 = jnp.dot(h1c, w4_ref[...], preferred_element_type=_F32) + b4_ref[...]
        dv2 = d_s[...] * v2
        dvf_s[...] = dv2
        dvb_s[...] = dv2.astype(_BF16)
        dvtb_s[...] = jnp.transpose(dv2).astype(_BF16)
        amax = ALPHA / jnp.max(lii_s[...])
        ts_s[...] = amax * jax.lax.dot_general(z_s[...], dv2, _TLHS,
                                               preferred_element_type=_F32)

    @pl.when(r == TOT - 1)
    def _():
        out_ref[...] = d_s[...] * core()


def _kbig(adj, adjU, h1, W2, s, ccol, u3, W4p, b4p):
    def _adj_idx(r):
        return (jnp.minimum(r, NB0 - 1), 0)

    def _panel_idx(r):
        return (jnp.where(r < NB0, 0, (r - NB0) % NBT),)

    return pl.pallas_call(
        _kbig_body,
        grid=(TOT,),
        in_specs=[
            pl.BlockSpec((BM, N), _adj_idx),
            pl.BlockSpec((BT, N), lambda r: _panel_idx(r) + (0,)),
            pl.BlockSpec((N, BT), lambda r: (0,) + _panel_idx(r)),
            pl.BlockSpec((N, H), lambda r: (0, 0)),
            pl.BlockSpec((H, H), lambda r: (0, 0)),
            pl.BlockSpec((N, 1), lambda r: (0, 0)),
            pl.BlockSpec((N, 1), lambda r: (0, 0)),
            pl.BlockSpec((N, H), lambda r: (0, 0)),
            pl.BlockSpec((H, H), lambda r: (0, 0)),
            pl.BlockSpec((1, H), lambda r: (0, 0)),
        ],
        out_specs=pl.BlockSpec((N, H), lambda r: (0, 0)),
        out_shape=jax.ShapeDtypeStruct((N, H), _F32),
        scratch_shapes=[
            pltpu.VMEM((N, H), _BF16),
            pltpu.VMEM((N, H), _F32),
            pltpu.VMEM((N, 1), _F32),
            pltpu.VMEM((1, H), _F32),
            pltpu.VMEM((N, 1), _F32),
            pltpu.VMEM((N, 1), _F32),
            pltpu.VMEM((N, H), _F32),
            pltpu.VMEM((N, H), _BF16),
            pltpu.VMEM((H, N), _BF16),
            pltpu.VMEM((H, H), _F32),
            pltpu.VMEM((NBT, BT, H), _F32),
            pltpu.VMEM((NBT, H, BT), _F32),
        ],
        compiler_params=_ARB,
    )(adj, adjU, adjU, h1, W2, s, ccol, u3, W4p, b4p)


def kernel(adj, x, W1, W2, W3, b3, W4, b4):
    nclass = W4.shape[1]
    b3r = b3.reshape(1, H).astype(_F32)
    W4p = jnp.zeros((H, H), _F32).at[:, :nclass].set(W4)
    b4p = jnp.zeros((1, H), _F32).at[0, :nclass].set(b4)

    hx, u3 = _k1(x, W1, W3, b3r)
    h1, s, cT, adjU = _k2(adj, hx)
    ccol = cT.T
    outf = _kbig(adj, adjU, h1, W2, s, ccol, u3, W4p, b4p)
    return outf[:, :nclass]


# speedup vs baseline: 2.4750x; 1.0295x over previous
import numpy as np
import jax
import jax.numpy as jnp
from jax.experimental import pallas as pl
from jax.experimental.pallas import tpu as pltpu

N = 4096
F = 256
H = 128
ALPHA = 0.8
BM = 256
BT = 512
NBT = N // BT
NB0 = N // BM
TOT = NB0 + 2 * NBT

_ARB = pltpu.CompilerParams(dimension_semantics=("arbitrary",))
_F32 = jnp.float32
_BF16 = jnp.bfloat16
_TLHS = (((0,), (0,)), ((), ()))


def _k2_body(adj_ref, x_ref, w1_ref, w2_ref, w3_ref, b3_ref,
             g1_ref, s_ref, ct_ref, au_ref, u3_ref, gx_s):
    i = pl.program_id(0)

    @pl.when(i == 0)
    def _():
        x = x_ref[...]
        w12 = jnp.dot(w1_ref[...], w2_ref[...], preferred_element_type=_F32)
        gx_s[...] = jnp.dot(x.astype(_BF16), w12.astype(_BF16),
                            preferred_element_type=_F32).astype(_BF16)
        u3_ref[...] = jnp.dot(x, w3_ref[...],
                              preferred_element_type=_F32) + b3_ref[...]

    a = adj_ref[...]
    g1 = jnp.dot(a.astype(_BF16), gx_s[...], preferred_element_type=_F32)
    g1_ref[...] = g1.astype(_BF16)
    rowg = i * BM + jax.lax.broadcasted_iota(jnp.int32, (BM, N), 0)
    colg = jax.lax.broadcasted_iota(jnp.int32, (BM, N), 1)
    am = jnp.where(colg > rowg, a, 0.0)
    au_ref[...] = am.astype(_BF16)
    s_ref[...] = jnp.sum(am, axis=1, keepdims=True)

    @pl.when(i == 0)
    def _():
        ct_ref[...] = jnp.zeros_like(ct_ref)

    ct_ref[...] += jnp.sum(am, axis=0, keepdims=True)


def _k2(adj, x, W1, W2, W3, b3r):
    return pl.pallas_call(
        _k2_body,
        grid=(NB0,),
        in_specs=[pl.BlockSpec((BM, N), lambda i: (i, 0)),
                  pl.BlockSpec((N, F), lambda i: (0, 0)),
                  pl.BlockSpec((F, H), lambda i: (0, 0)),
                  pl.BlockSpec((H, H), lambda i: (0, 0)),
                  pl.BlockSpec((F, H), lambda i: (0, 0)),
                  pl.BlockSpec((1, H), lambda i: (0, 0))],
        out_specs=[pl.BlockSpec((BM, H), lambda i: (i, 0)),
                   pl.BlockSpec((BM, 1), lambda i: (i, 0)),
                   pl.BlockSpec((1, N), lambda i: (0, 0)),
                   pl.BlockSpec((BM, N), lambda i: (i, 0)),
                   pl.BlockSpec((N, H), lambda i: (0, 0))],
        out_shape=[jax.ShapeDtypeStruct((N, H), _BF16),
                   jax.ShapeDtypeStruct((N, 1), _F32),
                   jax.ShapeDtypeStruct((1, N), _F32),
                   jax.ShapeDtypeStruct((N, N), _BF16),
                   jax.ShapeDtypeStruct((N, H), _F32)],
        scratch_shapes=[pltpu.VMEM((N, H), _BF16)],
        compiler_params=_ARB,
    )(adj, x, W1, W2, W3, b3r)


def _kbig_body(adj_ref, aurow_ref, aucol_ref, g1_ref, s_ref, c_ref,
               u3_ref, w4_ref, b4_ref, out_ref,
               z_s, lii_s, sc_s, d_s, liia_s, dvf_s, dvb_s, dvtb_s,
               ts_s, o1, o2t):
    r = pl.program_id(0)

    @pl.when(r == 0)
    def _():
        sc_s[...] = jnp.zeros_like(sc_s)

    @pl.when(r < NB0)
    def _():
        zb = jnp.maximum(jnp.dot(adj_ref[...].astype(_BF16), g1_ref[...],
                                 preferred_element_type=_F32), 0.0)
        z_s[pl.ds(r * BM, BM), :] = zb
        lii_s[pl.ds(r * BM, BM), :] = jnp.sum(zb * zb, axis=1, keepdims=True)
        sc_s[...] += jnp.sum(zb, axis=0, keepdims=True)

    @pl.when(r == NB0 - 1)
    def _():
        lii = lii_s[...]
        am = ALPHA / jnp.max(lii)
        z = z_s[...]
        zdots = jnp.sum(z * sc_s[...], axis=1, keepdims=True)
        rowsum = 1.0 + am * (zdots - lii) \
            + (1.0 - ALPHA) * (s_ref[...] + c_ref[...])
        d = jax.lax.rsqrt(rowsum)
        d_s[...] = d
        liia_s[...] = am * lii
        du3 = d * u3_ref[...]
        dvf_s[...] = du3
        dvb_s[...] = du3.astype(_BF16)
        dvtb_s[...] = jnp.transpose(du3).astype(_BF16)
        ts_s[...] = am * jax.lax.dot_general(z, du3, _TLHS,
                                             preferred_element_type=_F32)

    @pl.when(r >= NB0)
    def _():
        rr = (r - NB0) % NBT
        o1[rr] = jnp.dot(aurow_ref[...], dvb_s[...],
                         preferred_element_type=_F32)
        o2t[rr] = jnp.dot(dvtb_s[...], aucol_ref[...],
                          preferred_element_type=_F32)

    def core():
        dv = dvf_s[...]
        o2rows = jnp.concatenate(
            [jnp.transpose(o2t[cb]) for cb in range(NBT)], axis=0)
        return jnp.dot(z_s[...], ts_s[...], preferred_element_type=_F32) \
            - liia_s[...] * dv \
            + (1.0 - ALPHA) * (o1[...].reshape(N, H) + o2rows) \
            + dv

    @pl.when(r == NB0 + NBT - 1)
    def _():
        h1c = jnp.maximum(d_s[...] * core(), 0.0)
        v2 = jnp.dot(h1c, w4_ref[...], preferred_element_type=_F32) + b4_ref[...]
        dv2 = d_s[...] * v2
        dvf_s[...] = dv2
        dvb_s[...] = dv2.astype(_BF16)
        dvtb_s[...] = jnp.transpose(dv2).astype(_BF16)
        amax = ALPHA / jnp.max(lii_s[...])
        ts_s[...] = amax * jax.lax.dot_general(z_s[...], dv2, _TLHS,
                                               preferred_element_type=_F32)

    @pl.when(r == TOT - 1)
    def _():
        out_ref[...] = d_s[...] * core()


def _kbig(adj, adjU, g1, s, ccol, u3, W4p, b4p):
    def _adj_idx(r):
        return (jnp.minimum(r, NB0 - 1), 0)

    def _panel_idx(r):
        return (jnp.where(r < NB0, 0, (r - NB0) % NBT),)

    return pl.pallas_call(
        _kbig_body,
        grid=(TOT,),
        in_specs=[
            pl.BlockSpec((BM, N), _adj_idx),
            pl.BlockSpec((BT, N), lambda r: _panel_idx(r) + (0,)),
            pl.BlockSpec((N, BT), lambda r: (0,) + _panel_idx(r)),
            pl.BlockSpec((N, H), lambda r: (0, 0)),
            pl.BlockSpec((N, 1), lambda r: (0, 0)),
            pl.BlockSpec((N, 1), lambda r: (0, 0)),
            pl.BlockSpec((N, H), lambda r: (0, 0)),
            pl.BlockSpec((H, H), lambda r: (0, 0)),
            pl.BlockSpec((1, H), lambda r: (0, 0)),
        ],
        out_specs=pl.BlockSpec((N, H), lambda r: (0, 0)),
        out_shape=jax.ShapeDtypeStruct((N, H), _F32),
        scratch_shapes=[
            pltpu.VMEM((N, H), _F32),
            pltpu.VMEM((N, 1), _F32),
            pltpu.VMEM((1, H), _F32),
            pltpu.VMEM((N, 1), _F32),
            pltpu.VMEM((N, 1), _F32),
            pltpu.VMEM((N, H), _F32),
            pltpu.VMEM((N, H), _BF16),
            pltpu.VMEM((H, N), _BF16),
            pltpu.VMEM((H, H), _F32),
            pltpu.VMEM((NBT, BT, H), _F32),
            pltpu.VMEM((NBT, H, BT), _F32),
        ],
        compiler_params=_ARB,
    )(adj, adjU, adjU, g1, s, ccol, u3, W4p, b4p)


def kernel(adj, x, W1, W2, W3, b3, W4, b4):
    nclass = W4.shape[1]
    b3r = b3.reshape(1, H).astype(_F32)
    W4p = jnp.zeros((H, H), _F32).at[:, :nclass].set(W4)
    b4p = jnp.zeros((1, H), _F32).at[0, :nclass].set(b4)

    g1, s, cT, adjU, u3 = _k2(adj, x, W1, W2, W3, b3r)
    ccol = cT.T
    outf = _kbig(adj, adjU, g1, s, ccol, u3, W4p, b4p)
    return outf[:, :nclass]


# speedup vs baseline: 3.3541x; 1.3552x over previous
import numpy as np
import jax
import jax.numpy as jnp
from jax.experimental import pallas as pl
from jax.experimental.pallas import tpu as pltpu

N = 4096
F = 256
H = 128
ALPHA = 0.8
BM = 256
BT = 512
NBT = N // BT
NB0 = N // BM
TOT = 2 * NB0 + 2 * NBT

_ARB = pltpu.CompilerParams(dimension_semantics=("arbitrary",))
_F32 = jnp.float32
_BF16 = jnp.bfloat16
_TLHS = (((0,), (0,)), ((), ()))


def _k0_body(x_ref, w1_ref, w2_ref, w3_ref, b3_ref, gx_ref, u3_ref):
    x = x_ref[...]
    w12 = jnp.dot(w1_ref[...], w2_ref[...], preferred_element_type=_F32)
    gx = jnp.dot(x.astype(_BF16), w12.astype(_BF16),
                 preferred_element_type=_F32)
    gx_ref[...] = gx.astype(_BF16)
    u3_ref[...] = jnp.dot(x, w3_ref[...], preferred_element_type=_F32) + b3_ref[...]


def _k0(x, W1, W2, W3, b3r):
    return pl.pallas_call(
        _k0_body,
        out_shape=[jax.ShapeDtypeStruct((N, H), _BF16),
                   jax.ShapeDtypeStruct((N, H), _F32)],
    )(x, W1, W2, W3, b3r)


def _kmega_body(adj_ref, gx_ref, u3_ref, w4_ref, b4_ref, out_ref,
                g1_s, z_s, sc_s, ct_s, vec_s, dvf_s, dvb_s, dvtb_s,
                ts_s, o1, o2t, *au):
    r = pl.program_id(0)

    @pl.when(r < NB0)
    def _():
        i = r
        a = adj_ref[...]
        g1 = jnp.dot(a.astype(_BF16), gx_ref[...], preferred_element_type=_F32)
        g1_s[pl.ds(i * BM, BM), :] = g1.astype(_BF16)
        rowg = i * BM + jax.lax.broadcasted_iota(jnp.int32, (BM, N), 0)
        colg = jax.lax.broadcasted_iota(jnp.int32, (BM, N), 1)
        am = jnp.where(colg > rowg, a, 0.0)
        amb = am.astype(_BF16)
        for ct in range(NBT):
            @pl.when(i <= 2 * ct + 1)
            def _():
                au[ct][pl.ds(i * BM, BM), :] = amb[:, ct * BT:(ct + 1) * BT]
        vec_s[pl.ds(i * BM, BM), 0:1] = jnp.sum(am, axis=1, keepdims=True)

        @pl.when(i == 0)
        def _():
            ct_s[...] = jnp.zeros_like(ct_s)

        ct_s[...] += jnp.sum(am, axis=0, keepdims=True)

    @pl.when((r >= NB0) & (r < 2 * NB0))
    def _():
        i = r - NB0
        zb = jnp.maximum(jnp.dot(adj_ref[...].astype(_BF16), g1_s[...],
                                 preferred_element_type=_F32), 0.0)
        z_s[pl.ds(i * BM, BM), :] = zb
        vec_s[pl.ds(i * BM, BM), 1:2] = jnp.sum(zb * zb, axis=1, keepdims=True)

        @pl.when(i == 0)
        def _():
            sc_s[...] = jnp.zeros_like(sc_s)

        sc_s[...] += jnp.sum(zb, axis=0, keepdims=True)

    @pl.when(r == 2 * NB0 - 1)
    def _():
        lii = vec_s[:, 1:2]
        am = ALPHA / jnp.max(lii)
        z = z_s[...]
        zdots = jnp.sum(z * sc_s[...], axis=1, keepdims=True)
        ccol = jnp.transpose(ct_s[...])
        rowsum = 1.0 + am * (zdots - lii) \
            + (1.0 - ALPHA) * (vec_s[:, 0:1] + ccol)
        d = jax.lax.rsqrt(rowsum)
        vec_s[:, 2:3] = d
        vec_s[:, 3:4] = am * lii
        du3 = d * u3_ref[...]
        dvf_s[...] = du3
        dvb_s[...] = du3.astype(_BF16)
        dvtb_s[...] = jnp.transpose(du3).astype(_BF16)
        ts_s[...] = am * jax.lax.dot_general(z, du3, _TLHS,
                                             preferred_element_type=_F32)

    for phase in range(2):
        base = 2 * NB0 + phase * NBT
        for rr in range(NBT):
            @pl.when(r == base + rr)
            def _(rr=rr):
                dvb = dvb_s[...]
                acc = jnp.zeros((BT, H), _F32)
                for ct in range(rr, NBT):
                    acc += jnp.dot(au[ct][rr * BT:(rr + 1) * BT, :],
                                   dvb[ct * BT:(ct + 1) * BT, :],
                                   preferred_element_type=_F32)
                o1[rr] = acc
                o2t[rr] = jnp.dot(dvtb_s[:, :(rr + 1) * BT], au[rr][...],
                                  preferred_element_type=_F32)

    def core():
        dv = dvf_s[...]
        o2rows = jnp.concatenate(
            [jnp.transpose(o2t[cb]) for cb in range(NBT)], axis=0)
        return jnp.dot(z_s[...], ts_s[...], preferred_element_type=_F32) \
            - vec_s[:, 3:4] * dv \
            + (1.0 - ALPHA) * (o1[...].reshape(N, H) + o2rows) \
            + dv

    @pl.when(r == 2 * NB0 + NBT - 1)
    def _():
        d = vec_s[:, 2:3]
        h1c = jnp.maximum(d * core(), 0.0)
        v2 = jnp.dot(h1c, w4_ref[...], preferred_element_type=_F32) + b4_ref[...]
        dv2 = d * v2
        dvf_s[...] = dv2
        dvb_s[...] = dv2.astype(_BF16)
        dvtb_s[...] = jnp.transpose(dv2).astype(_BF16)
        amax = ALPHA / jnp.max(vec_s[:, 1:2])
        ts_s[...] = amax * jax.lax.dot_general(z_s[...], dv2, _TLHS,
                                               preferred_element_type=_F32)

    @pl.when(r == TOT - 1)
    def _():
        out_ref[...] = vec_s[:, 2:3] * core()


def _kmega(adj, gx, u3, W4p, b4p):
    def _adj_idx(r):
        return (jnp.where(r < NB0, r,
                          jnp.where(r < 2 * NB0, r - NB0, NB0 - 1)), 0)

    au_scratch = [pltpu.VMEM(((ct + 1) * BT, BT), _BF16) for ct in range(NBT)]
    return pl.pallas_call(
        _kmega_body,
        grid=(TOT,),
        in_specs=[
            pl.BlockSpec((BM, N), _adj_idx),
            pl.BlockSpec((N, H), lambda r: (0, 0)),
            pl.BlockSpec((N, H), lambda r: (0, 0)),
            pl.BlockSpec((H, H), lambda r: (0, 0)),
            pl.BlockSpec((1, H), lambda r: (0, 0)),
        ],
        out_specs=pl.BlockSpec((N, H), lambda r: (0, 0)),
        out_shape=jax.ShapeDtypeStruct((N, H), _F32),
        scratch_shapes=[
            pltpu.VMEM((N, H), _BF16),
            pltpu.VMEM((N, H), _F32),
            pltpu.VMEM((1, H), _F32),
            pltpu.VMEM((1, N), _F32),
            pltpu.VMEM((N, 8), _F32),
            pltpu.VMEM((N, H), _F32),
            pltpu.VMEM((N, H), _BF16),
            pltpu.VMEM((H, N), _BF16),
            pltpu.VMEM((H, H), _F32),
            pltpu.VMEM((NBT, BT, H), _F32),
            pltpu.VMEM((NBT, H, BT), _F32),
        ] + au_scratch,
        compiler_params=_ARB,
    )(adj, gx, u3, W4p, b4p)


def kernel(adj, x, W1, W2, W3, b3, W4, b4):
    nclass = W4.shape[1]
    b3r = b3.reshape(1, H).astype(_F32)
    W4p = jnp.zeros((H, H), _F32).at[:, :nclass].set(W4)
    b4p = jnp.zeros((1, H), _F32).at[0, :nclass].set(b4)

    gx, u3 = _k0(x, W1, W2, W3, b3r)
    outf = _kmega(adj, gx, u3, W4p, b4p)
    return outf[:, :nclass]


# speedup vs baseline: 3.6186x; 1.0789x over previous
import numpy as np
import jax
import jax.numpy as jnp
from jax.experimental import pallas as pl
from jax.experimental.pallas import tpu as pltpu

N = 4096
F = 256
H = 128
ALPHA = 0.8
BM = 512
BT = 512
NBT = N // BT
NB0 = N // BM
TOT = 2 * NB0 + 2 * NBT

_ARB = pltpu.CompilerParams(dimension_semantics=("arbitrary",))
_F32 = jnp.float32
_BF16 = jnp.bfloat16
_TLHS = (((0,), (0,)), ((), ()))


def _k0_body(x_ref, w1_ref, w2_ref, w3_ref, b3_ref, gx_ref, u3_ref):
    x = x_ref[...]
    w12 = jnp.dot(w1_ref[...], w2_ref[...], preferred_element_type=_F32)
    gx = jnp.dot(x.astype(_BF16), w12.astype(_BF16),
                 preferred_element_type=_F32)
    gx_ref[...] = gx.astype(_BF16)
    u3 = jnp.dot(x, w3_ref[...], preferred_element_type=_F32) + b3_ref[...]
    u3_ref[...] = u3.astype(_BF16)


def _k0(x, W1, W2, W3, b3r):
    return pl.pallas_call(
        _k0_body,
        out_shape=[jax.ShapeDtypeStruct((N, H), _BF16),
                   jax.ShapeDtypeStruct((N, H), _BF16)],
    )(x, W1, W2, W3, b3r)


def _kmega_body(adj_ref, gx_ref, u3_ref, w4_ref, b4_ref, out_ref,
                g1_s, z_s, sc_s, ct_s, vec_s, dvf_s, dvb_s, dvtb_s,
                ts_s, o1, o2t, *au):
    r = pl.program_id(0)

    @pl.when(r < NB0)
    def _():
        i = r
        a = adj_ref[...]
        g1 = jnp.dot(a.astype(_BF16), gx_ref[...], preferred_element_type=_F32)
        g1_s[pl.ds(i * BM, BM), :] = g1.astype(_BF16)
        rowg = i * BM + jax.lax.broadcasted_iota(jnp.int32, (BM, N), 0)
        colg = jax.lax.broadcasted_iota(jnp.int32, (BM, N), 1)
        am = jnp.where(colg > rowg, a, 0.0)
        amb = am.astype(_BF16)
        for ct in range(NBT):
            @pl.when(i * BM < (ct + 1) * BT)
            def _():
                au[ct][pl.ds(i * BM, BM), :] = amb[:, ct * BT:(ct + 1) * BT]
        vec_s[pl.ds(i * BM, BM), 0:1] = jnp.sum(am, axis=1, keepdims=True)

        @pl.when(i == 0)
        def _():
            ct_s[...] = jnp.zeros_like(ct_s)

        ct_s[...] += jnp.sum(am, axis=0, keepdims=True)

    @pl.when((r >= NB0) & (r < 2 * NB0))
    def _():
        i = r - NB0
        zb = jnp.maximum(jnp.dot(adj_ref[...].astype(_BF16), g1_s[...],
                                 preferred_element_type=_F32), 0.0)
        z_s[pl.ds(i * BM, BM), :] = zb.astype(_BF16)
        vec_s[pl.ds(i * BM, BM), 1:2] = jnp.sum(zb * zb, axis=1, keepdims=True)

        @pl.when(i == 0)
        def _():
            sc_s[...] = jnp.zeros_like(sc_s)

        sc_s[...] += jnp.sum(zb, axis=0, keepdims=True)

    @pl.when(r == 2 * NB0 - 1)
    def _():
        lii = vec_s[:, 1:2]
        am = ALPHA / jnp.max(lii)
        zdots = jnp.sum(z_s[...].astype(_F32) * sc_s[...], axis=1,
                        keepdims=True)
        ccol = jnp.transpose(ct_s[...])
        rowsum = 1.0 + am * (zdots - lii) \
            + (1.0 - ALPHA) * (vec_s[:, 0:1] + ccol)
        d = jax.lax.rsqrt(rowsum)
        vec_s[:, 2:3] = d
        vec_s[:, 3:4] = am * lii
        du3 = d * u3_ref[...].astype(_F32)
        dvf_s[...] = du3
        dvb = du3.astype(_BF16)
        dvb_s[...] = dvb
        dvtb_s[...] = jnp.transpose(du3).astype(_BF16)
        ts_s[...] = am * jax.lax.dot_general(z_s[...], dvb, _TLHS,
                                             preferred_element_type=_F32)

    for phase in range(2):
        base = 2 * NB0 + phase * NBT
        for rr in range(NBT):
            @pl.when(r == base + rr)
            def _(rr=rr):
                dvb = dvb_s[...]
                acc = jnp.zeros((BT, H), _F32)
                for ct in range(rr, NBT):
                    acc += jnp.dot(au[ct][rr * BT:(rr + 1) * BT, :],
                                   dvb[ct * BT:(ct + 1) * BT, :],
                                   preferred_element_type=_F32)
                o1[rr] = acc
                o2t[rr] = jnp.dot(dvtb_s[:, :(rr + 1) * BT], au[rr][...],
                                  preferred_element_type=_F32)

    def core():
        dv = dvf_s[...]
        o2rows = jnp.concatenate(
            [jnp.transpose(o2t[cb]) for cb in range(NBT)], axis=0)
        return jnp.dot(z_s[...], ts_s[...].astype(_BF16),
                       preferred_element_type=_F32) \
            - vec_s[:, 3:4] * dv \
            + (1.0 - ALPHA) * (o1[...].reshape(N, H) + o2rows) \
            + dv

    @pl.when(r == 2 * NB0 + NBT - 1)
    def _():
        d = vec_s[:, 2:3]
        h1c = jnp.maximum(d * core(), 0.0)
        v2 = jnp.dot(h1c, w4_ref[...], preferred_element_type=_F32) + b4_ref[...]
        dv2 = d * v2
        dvf_s[...] = dv2
        dv2b = dv2.astype(_BF16)
        dvb_s[...] = dv2b
        dvtb_s[...] = jnp.transpose(dv2).astype(_BF16)
        amax = ALPHA / jnp.max(vec_s[:, 1:2])
        ts_s[...] = amax * jax.lax.dot_general(z_s[...], dv2b, _TLHS,
                                               preferred_element_type=_F32)

    @pl.when(r == TOT - 1)
    def _():
        out_ref[...] = vec_s[:, 2:3] * core()


def _kmega(adj, gx, u3, W4p, b4p):
    def _adj_idx(r):
        return (jnp.where(r < NB0, r,
                          jnp.where(r < 2 * NB0, r - NB0, NB0 - 1)), 0)

    au_scratch = [pltpu.VMEM(((ct + 1) * BT, BT), _BF16) for ct in range(NBT)]
    return pl.pallas_call(
        _kmega_body,
        grid=(TOT,),
        in_specs=[
            pl.BlockSpec((BM, N), _adj_idx),
            pl.BlockSpec((N, H), lambda r: (0, 0)),
            pl.BlockSpec((N, H), lambda r: (0, 0)),
            pl.BlockSpec((H, H), lambda r: (0, 0)),
            pl.BlockSpec((1, H), lambda r: (0, 0)),
        ],
        out_specs=pl.BlockSpec((N, H), lambda r: (0, 0)),
        out_shape=jax.ShapeDtypeStruct((N, H), _F32),
        scratch_shapes=[
            pltpu.VMEM((N, H), _BF16),
            pltpu.VMEM((N, H), _BF16),
            pltpu.VMEM((1, H), _F32),
            pltpu.VMEM((1, N), _F32),
            pltpu.VMEM((N, 8), _F32),
            pltpu.VMEM((N, H), _F32),
            pltpu.VMEM((N, H), _BF16),
            pltpu.VMEM((H, N), _BF16),
            pltpu.VMEM((H, H), _F32),
            pltpu.VMEM((NBT, BT, H), _F32),
            pltpu.VMEM((NBT, H, BT), _F32),
        ] + au_scratch,
        compiler_params=_ARB,
    )(adj, gx, u3, W4p, b4p)


def kernel(adj, x, W1, W2, W3, b3, W4, b4):
    nclass = W4.shape[1]
    b3r = b3.reshape(1, H).astype(_F32)
    W4p = jnp.zeros((H, H), _F32).at[:, :nclass].set(W4)
    b4p = jnp.zeros((1, H), _F32).at[0, :nclass].set(b4)

    gx, u3 = _k0(x, W1, W2, W3, b3r)
    outf = _kmega(adj, gx, u3, W4p, b4p)
    return outf[:, :nclass]
